# baseline (speedup 1.0000x reference)
import functools

import jax
import jax.numpy as jnp
from jax import lax
from jax.experimental import pallas as pl
from jax.experimental.pallas import tpu as pltpu
from jax.experimental.pallas import tpu_sc as plsc


def _make_sc_gather(V, D, B, S, SP, DP, n_workers):
    BS = B * S
    assert BS % n_workers == 0
    b_per_w = BS // n_workers
    RB = 16
    C = RB * S
    NS = 2
    LOOKAHEAD = 1
    assert b_per_w % C == 0
    n_chunks = b_per_w // C
    rows_per_w = b_per_w // S

    mesh = plsc.VectorSubcoreMesh(core_axis_name="c", subcore_axis_name="s")

    @functools.partial(
        pl.kernel,
        mesh=mesh,
        compiler_params=pltpu.CompilerParams(use_tc_tiling_on_sc=False),
        out_type=jax.ShapeDtypeStruct((B, SP, DP), jnp.float32),
        scratch_types=[
            pltpu.VMEM((b_per_w,), jnp.int32),
            [pltpu.VMEM((C, D), jnp.float32) for _ in range(NS)],
            [pltpu.SemaphoreType.DMA for _ in range(NS)],
            [pltpu.SemaphoreType.DMA for _ in range(NS)],
        ],
    )
    def gather_kernel(table_hbm, idx_hbm, out_hbm, idx_v, rows, gsems, ssems):
        n_cores = lax.axis_size("c")
        wid = lax.axis_index("s") * n_cores + lax.axis_index("c")
        base = wid * b_per_w
        row_base = wid * rows_per_w

        pltpu.sync_copy(idx_hbm.at[pl.ds(base, b_per_w)], idx_v)

        def gather(c):
            b = c % NS
            pltpu.async_copy(
                table_hbm.at[idx_v.at[pl.ds(c * C, C)]], rows[b], gsems[b]
            )

        def wait_gather(c):
            b = c % NS
            pltpu.make_async_copy(
                table_hbm.at[idx_v.at[pl.ds(c * C, C)]], rows[b], gsems[b]
            ).wait()

        def scatter(c):
            b = c % NS
            for j in range(RB):
                pltpu.async_copy(
                    rows[b].at[pl.ds(j * S, S)],
                    out_hbm.at[row_base + c * RB + j].at[pl.ds(0, S), pl.ds(0, D)],
                    ssems[b],
                )

        def wait_scatter(c):
            b = c % NS
            for j in range(RB):
                pltpu.make_async_copy(
                    rows[b].at[pl.ds(j * S, S)],
                    out_hbm.at[row_base + c * RB + j].at[pl.ds(0, S), pl.ds(0, D)],
                    ssems[b],
                ).wait()

        for c in range(min(LOOKAHEAD, n_chunks)):
            gather(c)
        for c in range(n_chunks):
            wait_gather(c)
            scatter(c)
            f = c + LOOKAHEAD
            if f < n_chunks:
                if f >= NS:
                    wait_scatter(f - NS)
                gather(f)
        for c in range(max(0, n_chunks - NS), n_chunks):
            wait_scatter(c)

    return gather_kernel


def kernel(captions, table):
    B, S = captions.shape
    V, D = table.shape
    SP, DP = 56, 128
    flat_idx = captions.reshape(B * S).astype(jnp.int32)
    info = plsc.get_sparse_core_info()
    n_workers = info.num_cores * info.num_subcores
    out_p = _make_sc_gather(V, D, B, S, SP, DP, n_workers)(table, flat_idx)
    return out_p[:, :S, :D]


# speedup vs baseline: 1.0133x; 1.0133x over previous
import functools

import jax
import jax.numpy as jnp
from jax import lax
from jax.experimental import pallas as pl
from jax.experimental.pallas import tpu as pltpu
from jax.experimental.pallas import tpu_sc as plsc


def _make_sc_gather(V, D, B, S, SP, DP, n_workers):
    BS = B * S
    assert BS % n_workers == 0
    b_per_w = BS // n_workers
    RB = 4
    C = RB * S
    NS = 8
    LOOKAHEAD = 4
    assert b_per_w % C == 0
    n_chunks = b_per_w // C
    rows_per_w = b_per_w // S

    mesh = plsc.VectorSubcoreMesh(core_axis_name="c", subcore_axis_name="s")

    @functools.partial(
        pl.kernel,
        mesh=mesh,
        compiler_params=pltpu.CompilerParams(use_tc_tiling_on_sc=False),
        out_type=jax.ShapeDtypeStruct((B, SP, DP), jnp.float32),
        scratch_types=[
            pltpu.VMEM((b_per_w,), jnp.int32),
            [pltpu.VMEM((C, D), jnp.float32) for _ in range(NS)],
            [pltpu.SemaphoreType.DMA for _ in range(NS)],
            [pltpu.SemaphoreType.DMA for _ in range(NS)],
        ],
    )
    def gather_kernel(table_hbm, idx_hbm, out_hbm, idx_v, rows, gsems, ssems):
        n_cores = lax.axis_size("c")
        wid = lax.axis_index("s") * n_cores + lax.axis_index("c")
        base = wid * b_per_w
        row_base = wid * rows_per_w

        pltpu.sync_copy(idx_hbm.at[pl.ds(base, b_per_w)], idx_v)

        def gather(c):
            b = c % NS
            pltpu.async_copy(
                table_hbm.at[idx_v.at[pl.ds(c * C, C)]], rows[b], gsems[b]
            )

        def wait_gather(c):
            b = c % NS
            pltpu.make_async_copy(
                table_hbm.at[idx_v.at[pl.ds(c * C, C)]], rows[b], gsems[b]
            ).wait()

        def scatter(c):
            b = c % NS
            for j in range(RB):
                pltpu.async_copy(
                    rows[b].at[pl.ds(j * S, S)],
                    out_hbm.at[row_base + c * RB + j].at[pl.ds(0, S), pl.ds(0, D)],
                    ssems[b],
                )

        def wait_scatter(c):
            b = c % NS
            for j in range(RB):
                pltpu.make_async_copy(
                    rows[b].at[pl.ds(j * S, S)],
                    out_hbm.at[row_base + c * RB + j].at[pl.ds(0, S), pl.ds(0, D)],
                    ssems[b],
                ).wait()

        for c in range(min(LOOKAHEAD, n_chunks)):
            gather(c)
        for c in range(n_chunks):
            wait_gather(c)
            scatter(c)
            f = c + LOOKAHEAD
            if f < n_chunks:
                if f >= NS:
                    wait_scatter(f - NS)
                gather(f)
        for c in range(max(0, n_chunks - NS), n_chunks):
            wait_scatter(c)

    return gather_kernel


def kernel(captions, table):
    B, S = captions.shape
    V, D = table.shape
    SP, DP = 56, 128
    flat_idx = captions.reshape(B * S).astype(jnp.int32)
    info = plsc.get_sparse_core_info()
    n_workers = info.num_cores * info.num_subcores
    out_p = _make_sc_gather(V, D, B, S, SP, DP, n_workers)(table, flat_idx)
    return out_p[:, :S, :D]


# speedup vs baseline: 1.0194x; 1.0060x over previous
import functools

import jax
import jax.numpy as jnp
from jax import lax
from jax.experimental import pallas as pl
from jax.experimental.pallas import tpu as pltpu
from jax.experimental.pallas import tpu_sc as plsc


def _make_sc_gather(V, D, B, S, SP, DP, n_workers):
    BS = B * S
    assert BS % n_workers == 0
    b_per_w = BS // n_workers
    RB = 8
    C = RB * S
    NS = 4
    LOOKAHEAD = 3
    assert b_per_w % C == 0
    n_chunks = b_per_w // C
    rows_per_w = b_per_w // S

    mesh = plsc.VectorSubcoreMesh(core_axis_name="c", subcore_axis_name="s")

    @functools.partial(
        pl.kernel,
        mesh=mesh,
        compiler_params=pltpu.CompilerParams(use_tc_tiling_on_sc=False),
        out_type=jax.ShapeDtypeStruct((B, SP, DP), jnp.float32),
        scratch_types=[
            pltpu.VMEM((b_per_w,), jnp.int32),
            [pltpu.VMEM((C, D), jnp.float32) for _ in range(NS)],
            [pltpu.SemaphoreType.DMA for _ in range(NS)],
            [pltpu.SemaphoreType.DMA for _ in range(NS)],
        ],
    )
    def gather_kernel(table_hbm, idx_hbm, out_hbm, idx_v, rows, gsems, ssems):
        n_cores = lax.axis_size("c")
        wid = lax.axis_index("s") * n_cores + lax.axis_index("c")
        base = wid * b_per_w
        row_base = wid * rows_per_w

        pltpu.sync_copy(idx_hbm.at[pl.ds(base, b_per_w)], idx_v)

        def gather(c):
            b = c % NS
            pltpu.async_copy(
                table_hbm.at[idx_v.at[pl.ds(c * C, C)]], rows[b], gsems[b]
            )

        def wait_gather(c):
            b = c % NS
            pltpu.make_async_copy(
                table_hbm.at[idx_v.at[pl.ds(c * C, C)]], rows[b], gsems[b]
            ).wait()

        def scatter(c):
            b = c % NS
            for j in range(RB):
                pltpu.async_copy(
                    rows[b].at[pl.ds(j * S, S)],
                    out_hbm.at[row_base + c * RB + j].at[pl.ds(0, S), pl.ds(0, D)],
                    ssems[b],
                )

        def wait_scatter(c):
            b = c % NS
            for j in range(RB):
                pltpu.make_async_copy(
                    rows[b].at[pl.ds(j * S, S)],
                    out_hbm.at[row_base + c * RB + j].at[pl.ds(0, S), pl.ds(0, D)],
                    ssems[b],
                ).wait()

        for c in range(min(LOOKAHEAD, n_chunks)):
            gather(c)
        for c in range(n_chunks):
            wait_gather(c)
            scatter(c)
            f = c + LOOKAHEAD
            if f < n_chunks:
                if f >= NS:
                    wait_scatter(f - NS)
                gather(f)
        for c in range(max(0, n_chunks - NS), n_chunks):
            wait_scatter(c)

    return gather_kernel


def kernel(captions, table):
    B, S = captions.shape
    V, D = table.shape
    SP, DP = 56, 128
    flat_idx = captions.reshape(B * S).astype(jnp.int32)
    info = plsc.get_sparse_core_info()
    n_workers = info.num_cores * info.num_subcores
    out_p = _make_sc_gather(V, D, B, S, SP, DP, n_workers)(table, flat_idx)
    return out_p[:, :S, :D]
